# baseline (speedup 1.0000x reference)
import jax
import jax.numpy as jnp
import numpy as np
from jax import lax
from jax.experimental import pallas as pl
from jax.experimental.pallas import tpu as pltpu

BS, L, D, MEM_SIZE = 16, 8192, 768, 1024
N = BS * L
SCALE = np.sqrt(np.float32(D))
LB = 4096
PB = L // LB
DN11 = (((1,), (1,)), ((), ()))
LOG2L = 13


def _energy_kern(x_ref, wt_ref, bt_ref, wk_ref, qp_ref, e_ref):
    x2 = lax.dot_general(x_ref[...], wt_ref[...], DN11,
                         preferred_element_type=jnp.float32) + bt_ref[...]
    k = lax.dot_general(x2, wk_ref[...], DN11,
                        preferred_element_type=jnp.float32)
    e = lax.dot_general(qp_ref[0], k, DN11,
                        preferred_element_type=jnp.float32)
    e_ref[...] = (e / SCALE)[None]


def _qp_kern(q_ref, w_ref, o_ref):
    o_ref[...] = lax.dot_general(q_ref[...], w_ref[...], DN11,
                                 preferred_element_type=jnp.float32)


def _topk_kern(e_ref, tok_ref, msk_ref, out_tok_ref, out_msk_ref):
    e = e_ref[...]
    m = jnp.max(e, axis=-1, keepdims=True)
    ex = jnp.exp(e - m)
    s = jnp.sum(ex, axis=-1, keepdims=True)
    att = ex / s
    msk = msk_ref[...]
    att = jnp.where(msk == 0, 0.0, att)
    key = lax.bitcast_convert_type(att, jnp.int32)
    idx = lax.broadcasted_iota(jnp.int32, (BS, L), 1)
    payload = (idx << 18) | (tok_ref[...] << 1) | msk
    for k_ in range(1, LOG2L + 1):
        asc = ((idx >> k_) & 1) == 1 if k_ < LOG2L else jnp.zeros_like(idx, jnp.bool_)
        for j in range(k_ - 1, -1, -1):
            d = 1 << j
            is_lo = (idx & d) == 0
            key_up = pltpu_roll(key, -d)
            key_dn = pltpu_roll(key, d)
            pay_up = pltpu_roll(payload, -d)
            pay_dn = pltpu_roll(payload, d)
            pkey = jnp.where(is_lo, key_up, key_dn)
            ppay = jnp.where(is_lo, pay_up, pay_dn)
            self_better = (key > pkey) | ((key == pkey) & (payload < ppay))
            want_self = self_better ^ (~is_lo) ^ asc
            key = jnp.where(want_self, key, pkey)
            payload = jnp.where(want_self, payload, ppay)
    top = payload[:, :MEM_SIZE]
    out_tok_ref[...] = (top >> 1) & 0x1FFFF
    out_msk_ref[...] = top & 1


def pltpu_roll(x, shift):
    return jnp.roll(x, shift, axis=1)


GW = 8


def _gather_kern(idx_ref, *refs):
    in_refs = refs[:GW]
    out_ref = refs[GW]
    for j in range(GW):
        out_ref[0, j] = in_refs[j][0, 0]


def _emb_gather(emb_table, mem_input):
    emb3 = emb_table.reshape(emb_table.shape[0], 1, D)
    idx_flat = mem_input.reshape(N)
    specs = [
        pl.BlockSpec((1, 1, D), (lambda i, idx_ref, _j=j: (idx_ref[GW * i + _j], 0, 0)))
        for j in range(GW)
    ]
    out = pl.pallas_call(
        _gather_kern,
        grid_spec=pltpu.PrefetchScalarGridSpec(
            num_scalar_prefetch=1,
            grid=(N // GW,),
            in_specs=[specs[j] for j in range(GW)],
            out_specs=pl.BlockSpec((1, GW, D), lambda i, idx_ref: (i, 0, 0)),
        ),
        out_shape=jax.ShapeDtypeStruct((N // GW, GW, D), jnp.float32),
    )(idx_flat, *([emb3] * GW))
    return out.reshape(N, D)


def kernel(mem_input, mask, query, emb_table, W_trans, b_trans, Wq, Wk, Wv):
    x = _emb_gather(emb_table, mem_input)
    qp = pl.pallas_call(
        _qp_kern,
        in_specs=[pl.BlockSpec((BS, D), lambda: (0, 0)),
                  pl.BlockSpec((D, D), lambda: (0, 0))],
        out_specs=pl.BlockSpec((BS, D), lambda: (0, 0)),
        out_shape=jax.ShapeDtypeStruct((BS, D), jnp.float32),
    )(query, Wq)
    e = pl.pallas_call(
        _energy_kern,
        grid=(N // LB,),
        in_specs=[
            pl.BlockSpec((LB, D), lambda i: (i, 0)),
            pl.BlockSpec((D, D), lambda i: (0, 0)),
            pl.BlockSpec((1, D), lambda i: (0, 0)),
            pl.BlockSpec((D, D), lambda i: (0, 0)),
            pl.BlockSpec((1, 1, D), lambda i: (i // PB, 0, 0)),
        ],
        out_specs=pl.BlockSpec((1, 1, LB), lambda i: (i, 0, 0)),
        out_shape=jax.ShapeDtypeStruct((N // LB, 1, LB), jnp.float32),
    )(x, W_trans, b_trans.reshape(1, D), Wk, qp.reshape(BS, 1, D))
    e = e.reshape(BS, L)
    mem_output, mask_out = pl.pallas_call(
        _topk_kern,
        in_specs=[pl.BlockSpec((BS, L), lambda: (0, 0)),
                  pl.BlockSpec((BS, L), lambda: (0, 0)),
                  pl.BlockSpec((BS, L), lambda: (0, 0))],
        out_specs=[pl.BlockSpec((BS, MEM_SIZE), lambda: (0, 0)),
                   pl.BlockSpec((BS, MEM_SIZE), lambda: (0, 0))],
        out_shape=[jax.ShapeDtypeStruct((BS, MEM_SIZE), jnp.int32),
                   jax.ShapeDtypeStruct((BS, MEM_SIZE), jnp.int32)],
    )(e, mem_input, mask)
    return (mem_output, mask_out)


# speedup vs baseline: 8.6698x; 8.6698x over previous
import functools

import jax
import jax.numpy as jnp
import numpy as np
from jax import lax
from jax.experimental import pallas as pl
from jax.experimental.pallas import tpu as pltpu
from jax.experimental.pallas import tpu_sc as plsc

BS, L, D, MEM_SIZE = 16, 8192, 768, 1024
N = BS * L
SCALE = np.sqrt(np.float32(D))
LB = 4096
PB = L // LB
DN11 = (((1,), (1,)), ((), ()))
LOG2L = 13


def _energy_kern(x_ref, wt_ref, bt_ref, wk_ref, qp_ref, e_ref):
    x2 = lax.dot_general(x_ref[...], wt_ref[...], DN11,
                         preferred_element_type=jnp.float32) + bt_ref[...]
    k = lax.dot_general(x2, wk_ref[...], DN11,
                        preferred_element_type=jnp.float32)
    e = lax.dot_general(qp_ref[0], k, DN11,
                        preferred_element_type=jnp.float32)
    e_ref[...] = (e / SCALE)[None]


def _qp_kern(q_ref, w_ref, o_ref):
    o_ref[...] = lax.dot_general(q_ref[...], w_ref[...], DN11,
                                 preferred_element_type=jnp.float32)


def _topk_kern(e_ref, tok_ref, msk_ref, out_tok_ref, out_msk_ref):
    e = e_ref[...]
    m = jnp.max(e, axis=-1, keepdims=True)
    ex = jnp.exp(e - m)
    s = jnp.sum(ex, axis=-1, keepdims=True)
    att = ex / s
    msk = msk_ref[...]
    att = jnp.where(msk == 0, 0.0, att)
    key = lax.bitcast_convert_type(att, jnp.int32)
    idx = lax.broadcasted_iota(jnp.int32, (BS, L), 1)
    payload = (idx << 18) | (tok_ref[...] << 1) | msk
    for k_ in range(1, LOG2L + 1):
        asc = ((idx >> k_) & 1) == 1 if k_ < LOG2L else jnp.zeros_like(idx, jnp.bool_)
        for j in range(k_ - 1, -1, -1):
            d = 1 << j
            is_lo = (idx & d) == 0
            key_up = pltpu_roll(key, -d)
            key_dn = pltpu_roll(key, d)
            pay_up = pltpu_roll(payload, -d)
            pay_dn = pltpu_roll(payload, d)
            pkey = jnp.where(is_lo, key_up, key_dn)
            ppay = jnp.where(is_lo, pay_up, pay_dn)
            self_better = (key > pkey) | ((key == pkey) & (payload < ppay))
            want_self = self_better ^ (~is_lo) ^ asc
            key = jnp.where(want_self, key, pkey)
            payload = jnp.where(want_self, payload, ppay)
    top = payload[:, :MEM_SIZE]
    out_tok_ref[...] = (top >> 1) & 0x1FFFF
    out_msk_ref[...] = top & 1


def pltpu_roll(x, shift):
    return jnp.roll(x, shift, axis=1)


_NC, _NS = 2, 16
_NW = _NC * _NS
_BPW = N // _NW
_CH = 16


def _sc_gather_body(table_hbm, idx_hbm, out_hbm, idx_v, rows_v, sem):
    wid = lax.axis_index("s") * _NC + lax.axis_index("c")
    base = wid * _BPW

    @pl.loop(0, _BPW // _CH)
    def _chunk(i):
        off = base + i * _CH
        pltpu.sync_copy(idx_hbm.at[pl.ds(off, _CH)], idx_v)
        pltpu.async_copy(table_hbm.at[idx_v], rows_v, sem).wait()
        pltpu.sync_copy(rows_v, out_hbm.at[pl.ds(off, _CH)])


def _emb_gather(emb_table, mem_input):
    idx_flat = mem_input.reshape(N)
    mesh = plsc.VectorSubcoreMesh(core_axis_name="c", subcore_axis_name="s")
    fn = functools.partial(
        pl.kernel, mesh=mesh,
        out_type=jax.ShapeDtypeStruct((N, D), jnp.float32),
        scratch_types=[
            pltpu.VMEM((_CH,), jnp.int32),
            pltpu.VMEM((_CH, D), jnp.float32),
            pltpu.SemaphoreType.DMA,
        ],
    )(_sc_gather_body)
    return fn(emb_table, idx_flat)


def kernel(mem_input, mask, query, emb_table, W_trans, b_trans, Wq, Wk, Wv):
    x = _emb_gather(emb_table, mem_input)
    qp = pl.pallas_call(
        _qp_kern,
        in_specs=[pl.BlockSpec((BS, D), lambda: (0, 0)),
                  pl.BlockSpec((D, D), lambda: (0, 0))],
        out_specs=pl.BlockSpec((BS, D), lambda: (0, 0)),
        out_shape=jax.ShapeDtypeStruct((BS, D), jnp.float32),
    )(query, Wq)
    e = pl.pallas_call(
        _energy_kern,
        grid=(N // LB,),
        in_specs=[
            pl.BlockSpec((LB, D), lambda i: (i, 0)),
            pl.BlockSpec((D, D), lambda i: (0, 0)),
            pl.BlockSpec((1, D), lambda i: (0, 0)),
            pl.BlockSpec((D, D), lambda i: (0, 0)),
            pl.BlockSpec((1, 1, D), lambda i: (i // PB, 0, 0)),
        ],
        out_specs=pl.BlockSpec((1, 1, LB), lambda i: (i, 0, 0)),
        out_shape=jax.ShapeDtypeStruct((N // LB, 1, LB), jnp.float32),
    )(x, W_trans, b_trans.reshape(1, D), Wk, qp.reshape(BS, 1, D))
    e = e.reshape(BS, L)
    mem_output, mask_out = pl.pallas_call(
        _topk_kern,
        in_specs=[pl.BlockSpec((BS, L), lambda: (0, 0)),
                  pl.BlockSpec((BS, L), lambda: (0, 0)),
                  pl.BlockSpec((BS, L), lambda: (0, 0))],
        out_specs=[pl.BlockSpec((BS, MEM_SIZE), lambda: (0, 0)),
                   pl.BlockSpec((BS, MEM_SIZE), lambda: (0, 0))],
        out_shape=[jax.ShapeDtypeStruct((BS, MEM_SIZE), jnp.int32),
                   jax.ShapeDtypeStruct((BS, MEM_SIZE), jnp.int32)],
    )(e, mem_input, mask)
    return (mem_output, mask_out)
